# baseline (speedup 1.0000x reference)
import functools

import jax
import jax.numpy as jnp
from jax import lax
from jax.experimental import pallas as pl
from jax.experimental.pallas import tpu as pltpu
from jax.experimental.pallas import tpu_sc as plsc


def _proj_body(x_ref, wn_ref, bn_ref, ws_ref, bs_ref, y_ref, sf_ref):
    xx = x_ref[...]
    dn = (((1,), (1,)), ((), ()))
    y_ref[...] = lax.dot_general(xx, wn_ref[...], dn,
                                 preferred_element_type=jnp.float32) + bn_ref[...]
    sf_ref[...] = lax.dot_general(xx, ws_ref[...], dn,
                                  preferred_element_type=jnp.float32) + bs_ref[...]


def _proj(x, wn, bn, ws, bs, bn_rows):
    n, c = x.shape
    grid = n // bn_rows
    full = lambda i: (0, 0)
    return pl.pallas_call(
        _proj_body,
        grid=(grid,),
        in_specs=[
            pl.BlockSpec((bn_rows, c), lambda i: (i, 0)),
            pl.BlockSpec((c, c), full),
            pl.BlockSpec((1, c), full),
            pl.BlockSpec((c, c), full),
            pl.BlockSpec((1, c), full),
        ],
        out_specs=[
            pl.BlockSpec((bn_rows, c), lambda i: (i, 0)),
            pl.BlockSpec((bn_rows, c), lambda i: (i, 0)),
        ],
        out_shape=[
            jax.ShapeDtypeStruct((n, c), jnp.float32),
            jax.ShapeDtypeStruct((n, c), jnp.float32),
        ],
    )(x, wn, bn.reshape(1, c), ws, bs.reshape(1, c))


_NW = 32
_L = 16
_CH = 640
_BG = 32
_NS = 3


def _seg_reduce(y, pe, n, e, c):
    npt = ((n + _NW - 1) // _NW + 7) // 8 * 8
    nch = e // _CH
    assert nch * _CH == e and nch % 2 == 0
    lst = _CH + 2 * _BG
    fb = c // _L

    mesh = plsc.VectorSubcoreMesh(core_axis_name="c", subcore_axis_name="s",
                                  num_cores=2, num_subcores=16)

    @functools.partial(
        pl.kernel,
        mesh=mesh,
        compiler_params=pltpu.CompilerParams(needs_layout_passes=False),
        out_type=(
            jax.ShapeDtypeStruct((n * (c // 2),), jnp.int32),
            jax.ShapeDtypeStruct((n * (c // 2),), jnp.int32),
            jax.ShapeDtypeStruct((n * (c // 2),), jnp.float32),
            jax.ShapeDtypeStruct((n * (c // 2),), jnp.float32),
            jax.ShapeDtypeStruct((n,), jnp.float32),
        ),
        scratch_types=[
            pltpu.VMEM((npt * (c // 2),), jnp.int32),
            pltpu.VMEM((npt * (c // 2),), jnp.int32),
            pltpu.VMEM((npt * (c // 2),), jnp.float32),
            pltpu.VMEM((npt * (c // 2),), jnp.float32),
            pltpu.VMEM((npt + _L,), jnp.float32),
            pltpu.VMEM((_CH,), jnp.int32),
            pltpu.VMEM((_CH,), jnp.int32),
            pltpu.VMEM((lst,), jnp.int32),
            pltpu.VMEM((_NS, _BG), jnp.int32),
            pltpu.VMEM((_NS * _BG + _L,), jnp.int32),
            pltpu.VMEM((_NS, _BG, c), jnp.float32),
            pltpu.SemaphoreType.DMA,
            pltpu.SemaphoreType.DMA,
            pltpu.SemaphoreType.DMA,
            pltpu.SemaphoreType.DMA,
        ],
    )
    def k(y_hbm, pe_hbm, mm0_hbm, mm1_hbm, sm0_hbm, sm1_hbm, cnt_hbm,
          accmn0, accmn1, accs0, accs1, acnt, peA, peB, clist,
          gidx, gdst, ring,
          ssem, rsem0, rsem1, rsem2):
        cid = lax.axis_index("c")
        sid = lax.axis_index("s")
        wid = sid * 2 + cid
        lo = wid * npt
        hi = jnp.minimum(lo + npt, n)
        iota = lax.iota(jnp.int32, _L)
        one0 = jnp.where(iota == 0, 1.0, 0.0).astype(jnp.float32)
        ones = jnp.ones((_L,), jnp.float32)
        lane0 = iota == 0
        hi_mask = jnp.full((_L,), -65536, jnp.int32)
        half = jnp.full((_L,), 0x8000, jnp.int32)
        mninit = jnp.full((_L,), -8355968, jnp.int32)
        zero = jnp.zeros((_L,), jnp.float32)

        hc = c // 2

        def init_row(r, _):
            for f in range(fb // 2):
                accmn0[pl.ds(r * hc + f * _L, _L)] = mninit
                accmn1[pl.ds(r * hc + f * _L, _L)] = mninit
                accs0[pl.ds(r * hc + f * _L, _L)] = zero
                accs1[pl.ds(r * hc + f * _L, _L)] = zero
            return 0
        lax.fori_loop(0, npt, init_row, 0)

        def init_cnt(kk, _):
            acnt[pl.ds(kk * _L, _L)] = zero
            return 0
        lax.fori_loop(0, (npt + _L) // _L, init_cnt, 0)
        z16i = jnp.zeros((_L,), jnp.int32)
        for j in range(2 * _BG // _L):
            clist[pl.ds(j * _L, _L)] = z16i
        gdst[pl.ds(_NS * _BG, _L)] = z16i

        def issue_big(start, slot, rsem):
            for j in range(_BG // _L):
                v = clist[pl.ds(start + j * _L, _L)]
                gidx[slot, pl.ds(j * _L, _L)] = v >> 16
                gdst[pl.ds(slot * _BG + j * _L, _L)] = (v & 0xFFFF) - lo
            pltpu.async_copy(y_hbm.at[gidx.at[slot]], ring.at[slot], rsem)

        def drain_big(slot, rsem, count):
            pltpu.make_async_copy(y_hbm.at[gidx.at[slot]],
                                  ring.at[slot], rsem).wait()

            def fold(accmn, accs, iv, ev, f, h):
                colv = (h * (fb // 2) + f) * _L + iota
                rv = plsc.load_gather(ring.at[slot], [ev, colv])
                u = plsc.load_gather(accmn, [iv])
                mx = jnp.maximum(plsc.bitcast(u & hi_mask, jnp.float32), rv)
                mn = jnp.minimum(plsc.bitcast(u << 16, jnp.float32), rv)
                plsc.store_scatter(accmn, [iv], (
                    ((plsc.bitcast(mx, jnp.int32) + half) & hi_mask)
                    | lax.shift_right_logical(
                        plsc.bitcast(mn, jnp.int32) + half, 16)))
                plsc.addupdate_scatter(accs, [iv], rv)

            def accum_edge(ei, _):
                dv = plsc.load_gather(
                    gdst, [jnp.full((_L,), slot * _BG + ei, jnp.int32)])
                base = dv * hc + iota
                ev = jnp.full((_L,), ei, jnp.int32)
                for f in range(fb // 2):
                    iv = base + f * _L
                    fold(accmn0, accs0, iv, ev, f, 0)
                    fold(accmn1, accs1, iv, ev, f, 1)
                plsc.addupdate_scatter(acnt, [dv], ones, mask=lane0)
                return 0
            lax.fori_loop(0, count, accum_edge, 0)

        def issue_slot(start, k):
            r = k % _NS

            @pl.when(r == 0)
            def _():
                issue_big(start, 0, rsem0)

            @pl.when(r == 1)
            def _():
                issue_big(start, 1, rsem1)

            @pl.when(r == 2)
            def _():
                issue_big(start, 2, rsem2)

        def drain_slot(k, count):
            r = k % _NS

            @pl.when(r == 0)
            def _():
                drain_big(0, rsem0, count)

            @pl.when(r == 1)
            def _():
                drain_big(1, rsem1, count)

            @pl.when(r == 2)
            def _():
                drain_big(2, rsem2, count)

        def pump(ptr, bi):
            nbig = ptr // _BG

            def step(j, b):
                drain_slot(b - 3, jnp.where(b - 3 < _NS, 0, _BG).astype(jnp.int32))
                issue_slot(j * _BG, b)
                return b + 1
            bi = lax.fori_loop(0, nbig, step, bi)
            rem = ptr - nbig * _BG
            for j in range(_BG // _L):
                v = clist[pl.ds(nbig * _BG + j * _L, _L)]
                plsc.store_compressed(clist.at[pl.ds(j * _L, _L)], v,
                                      mask=iota < (rem - j * _L))
            return rem, bi

        def scan_chunk(pev, ptr):
            def scan(v, p):
                w = pev[pl.ds(v * _L, _L)]
                d = w & 0xFFFF
                m = (d >= lo) & (d < hi)
                plsc.store_compressed(clist.at[pl.ds(p, _L)], w, mask=m)
                return p + plsc.all_reduce_population_count(m)[0]
            return lax.fori_loop(0, _CH // _L, scan, ptr)

        def stage(ci, pev):
            pltpu.async_copy(pe_hbm.at[pl.ds(ci * _CH, _CH)], pev, ssem)

        def stage_wait(pev):
            pltpu.make_async_copy(pe_hbm.at[pl.ds(0, _CH)], pev, ssem).wait()

        stage(0, peA)
        issue_big(0, 0, rsem0)
        issue_big(0, 1, rsem1)
        issue_big(0, 2, rsem2)

        def pair(pi, carry):
            ptr, bi = carry
            ci = pi * 2
            stage_wait(peA)
            stage(ci + 1, peB)
            ptr = scan_chunk(peA, ptr)
            ptr, bi = pump(ptr, bi)
            stage_wait(peB)

            @pl.when(ci + 2 < nch)
            def _():
                stage(ci + 2, peA)
            ptr = scan_chunk(peB, ptr)
            ptr, bi = pump(ptr, bi)
            return (ptr, bi)

        ptr, bi = lax.fori_loop(0, nch // 2, pair,
                                (jnp.int32(0), jnp.int32(_NS)))

        drain_slot(bi - 3, jnp.where(bi - 3 < _NS, 0, _BG).astype(jnp.int32))
        drain_slot(bi - 2, jnp.where(bi - 2 < _NS, 0, _BG).astype(jnp.int32))
        drain_slot(bi - 1, jnp.where(bi - 1 < _NS, 0, _BG).astype(jnp.int32))
        issue_slot(0, bi)
        drain_slot(bi, ptr)

        nlast = n - (_NW - 1) * npt

        @pl.when(wid < _NW - 1)
        def _():
            pltpu.sync_copy(accmn0, mm0_hbm.at[pl.ds(lo * hc, npt * hc)])
            pltpu.sync_copy(accmn1, mm1_hbm.at[pl.ds(lo * hc, npt * hc)])
            pltpu.sync_copy(accs0, sm0_hbm.at[pl.ds(lo * hc, npt * hc)])
            pltpu.sync_copy(accs1, sm1_hbm.at[pl.ds(lo * hc, npt * hc)])
            pltpu.sync_copy(acnt.at[pl.ds(0, npt)], cnt_hbm.at[pl.ds(lo, npt)])

        @pl.when(wid == _NW - 1)
        def _():
            pltpu.sync_copy(accmn0.at[pl.ds(0, nlast * hc)],
                            mm0_hbm.at[pl.ds(lo * hc, nlast * hc)])
            pltpu.sync_copy(accmn1.at[pl.ds(0, nlast * hc)],
                            mm1_hbm.at[pl.ds(lo * hc, nlast * hc)])
            pltpu.sync_copy(accs0.at[pl.ds(0, nlast * hc)],
                            sm0_hbm.at[pl.ds(lo * hc, nlast * hc)])
            pltpu.sync_copy(accs1.at[pl.ds(0, nlast * hc)],
                            sm1_hbm.at[pl.ds(lo * hc, nlast * hc)])
            pltpu.sync_copy(acnt.at[pl.ds(0, nlast)], cnt_hbm.at[pl.ds(lo, nlast)])

    return k(y, pe)


def _attn_body(sf_ref, mm0_ref, mm1_ref, sm0_ref, sm1_ref, cnt_ref,
               ipw_ref, ipb_ref, opw_ref, opb_ref, out_ref):
    c = sf_ref.shape[1]
    sf = sf_ref[...]
    cnt = cnt_ref[...]
    has = cnt > 0.0
    u = jnp.concatenate([mm0_ref[...], mm1_ref[...]], axis=1)
    mx = jnp.where(has, lax.bitcast_convert_type(u & (-65536), jnp.float32), 0.0)
    mn = jnp.where(has, lax.bitcast_convert_type(u << 16, jnp.float32), 0.0)
    sm = jnp.where(has, jnp.concatenate([sm0_ref[...], sm1_ref[...]], axis=1), 0.0)
    mean = sm / jnp.maximum(cnt, 1.0)
    tokens = [sf, mx, mn, sm, mean]

    ipw = ipw_ref[...]
    ipb = ipb_ref[...]
    dn = (((1,), (1,)), ((), ()))
    big = jnp.concatenate(tokens, axis=0)
    qkv = lax.dot_general(big, ipw, dn, preferred_element_type=jnp.float32) + ipb
    b = sf.shape[0]
    q = [qkv[l * b:(l + 1) * b, 0:c] for l in range(5)]
    k = [qkv[l * b:(l + 1) * b, c:2 * c] for l in range(5)]
    v = [qkv[l * b:(l + 1) * b, 2 * c:3 * c] for l in range(5)]

    scale = 1.0 / jnp.sqrt(jnp.float32(c))
    s = [[jnp.sum(q[l] * k[m], axis=1, keepdims=True) * scale
          for m in range(5)] for l in range(5)]
    w = [jnp.zeros((b, 1), jnp.float32) for _ in range(5)]
    for l in range(5):
        smax = s[l][0]
        for m in range(1, 5):
            smax = jnp.maximum(smax, s[l][m])
        ex = [jnp.exp(s[l][m] - smax) for m in range(5)]
        z = ex[0] + ex[1] + ex[2] + ex[3] + ex[4]
        for m in range(5):
            w[m] = w[m] + ex[m] / z
    ctx = (w[0] * v[0] + w[1] * v[1] + w[2] * v[2] + w[3] * v[3] + w[4] * v[4]) * 0.2
    out = lax.dot_general(ctx, opw_ref[...], dn,
                          preferred_element_type=jnp.float32) + opb_ref[...]
    out_ref[...] = sf + out


def _attention(sf, mm0, mm1, sm0, sm1, cnt, ipw, ipb, opw, opb, bn_rows):
    n, c = sf.shape
    grid = n // bn_rows
    full = lambda i: (0, 0)
    blk = pl.BlockSpec((bn_rows, c), lambda i: (i, 0))
    half_blk = pl.BlockSpec((bn_rows, c // 2), lambda i: (i, 0))
    return pl.pallas_call(
        _attn_body,
        grid=(grid,),
        in_specs=[
            blk, half_blk, half_blk, half_blk, half_blk,
            pl.BlockSpec((bn_rows, 1), lambda i: (i, 0)),
            pl.BlockSpec((3 * c, c), full),
            pl.BlockSpec((1, 3 * c), full),
            pl.BlockSpec((c, c), full),
            pl.BlockSpec((1, c), full),
        ],
        out_specs=blk,
        out_shape=jax.ShapeDtypeStruct((n, c), jnp.float32),
    )(sf, mm0, mm1, sm0, sm1, cnt.reshape(n, 1), ipw, ipb.reshape(1, 3 * c),
      opw, opb.reshape(1, c))


def kernel(x, W_neighbor, b_neighbor, W_self, b_self, in_proj_w, in_proj_b,
           out_proj_w, out_proj_b, edge_index):
    n, c = x.shape
    e = edge_index.shape[1]
    pe = (edge_index[0] << 16) | edge_index[1]

    bn_rows = 400 if n % 400 == 0 else n
    y, sf = _proj(x, W_neighbor, b_neighbor, W_self, b_self, bn_rows)
    mm0, mm1, sm0, sm1, cnt = _seg_reduce(y, pe, n, e, c)
    hc = c // 2
    return _attention(sf, mm0.reshape(n, hc), mm1.reshape(n, hc),
                      sm0.reshape(n, hc), sm1.reshape(n, hc), cnt,
                      in_proj_w, in_proj_b, out_proj_w, out_proj_b, bn_rows)


# speedup vs baseline: 1.0798x; 1.0798x over previous
import functools

import jax
import jax.numpy as jnp
from jax import lax
from jax.experimental import pallas as pl
from jax.experimental.pallas import tpu as pltpu
from jax.experimental.pallas import tpu_sc as plsc


def _proj_body(x_ref, wn_ref, bn_ref, ws_ref, bs_ref, y_ref, sf_ref):
    xx = x_ref[...]
    dn = (((1,), (1,)), ((), ()))
    y_ref[...] = lax.dot_general(xx, wn_ref[...], dn,
                                 preferred_element_type=jnp.float32) + bn_ref[...]
    sf_ref[...] = lax.dot_general(xx, ws_ref[...], dn,
                                  preferred_element_type=jnp.float32) + bs_ref[...]


def _proj(x, wn, bn, ws, bs, bn_rows):
    n, c = x.shape
    grid = n // bn_rows
    full = lambda i: (0, 0)
    return pl.pallas_call(
        _proj_body,
        grid=(grid,),
        in_specs=[
            pl.BlockSpec((bn_rows, c), lambda i: (i, 0)),
            pl.BlockSpec((c, c), full),
            pl.BlockSpec((1, c), full),
            pl.BlockSpec((c, c), full),
            pl.BlockSpec((1, c), full),
        ],
        out_specs=[
            pl.BlockSpec((bn_rows, c), lambda i: (i, 0)),
            pl.BlockSpec((bn_rows, c), lambda i: (i, 0)),
        ],
        out_shape=[
            jax.ShapeDtypeStruct((n, c), jnp.float32),
            jax.ShapeDtypeStruct((n, c), jnp.float32),
        ],
    )(x, wn, bn.reshape(1, c), ws, bs.reshape(1, c))


_NW = 32
_L = 16
_CH = 640
_BG = 32
_NS = 3


def _seg_reduce(y, pe, n, e, c):
    npt = ((n + _NW - 1) // _NW + 7) // 8 * 8
    nch = e // _CH
    assert nch * _CH == e and nch % 2 == 0
    lst = _CH + 2 * _BG
    fb = c // _L

    mesh = plsc.VectorSubcoreMesh(core_axis_name="c", subcore_axis_name="s",
                                  num_cores=2, num_subcores=16)

    @functools.partial(
        pl.kernel,
        mesh=mesh,
        compiler_params=pltpu.CompilerParams(needs_layout_passes=False),
        out_type=(
            jax.ShapeDtypeStruct((n, c), jnp.int32),
            jax.ShapeDtypeStruct((n, c), jnp.float32),
            jax.ShapeDtypeStruct((n,), jnp.float32),
        ),
        scratch_types=[
            pltpu.VMEM((npt, c), jnp.int32),
            pltpu.VMEM((npt, c), jnp.float32),
            pltpu.VMEM((npt + _L,), jnp.float32),
            pltpu.VMEM((_CH,), jnp.int32),
            pltpu.VMEM((_CH,), jnp.int32),
            pltpu.VMEM((lst,), jnp.int32),
            pltpu.VMEM((_NS, _BG), jnp.int32),
            pltpu.VMEM((_NS * _BG + _L,), jnp.int32),
            pltpu.VMEM((_NS, _BG, c), jnp.float32),
            pltpu.SemaphoreType.DMA,
            pltpu.SemaphoreType.DMA,
            pltpu.SemaphoreType.DMA,
            pltpu.SemaphoreType.DMA,
        ],
    )
    def k(y_hbm, pe_hbm, mm_hbm, sm_hbm, cnt_hbm,
          accmn, accs, acnt, peA, peB, clist,
          gidx, gdst, ring,
          ssem, rsem0, rsem1, rsem2):
        cid = lax.axis_index("c")
        sid = lax.axis_index("s")
        wid = sid * 2 + cid
        lo = wid * npt
        hi = jnp.minimum(lo + npt, n)
        iota = lax.iota(jnp.int32, _L)
        one0 = jnp.where(iota == 0, 1.0, 0.0).astype(jnp.float32)
        hi_mask = jnp.full((_L,), -65536, jnp.int32)
        half = jnp.full((_L,), 0x8000, jnp.int32)
        mninit = jnp.full((_L,), -8355968, jnp.int32)
        zero = jnp.zeros((_L,), jnp.float32)

        def init_row(r, _):
            for f in range(fb):
                accmn[r, pl.ds(f * _L, _L)] = mninit
                accs[r, pl.ds(f * _L, _L)] = zero
            return 0
        lax.fori_loop(0, npt, init_row, 0)

        def init_cnt(kk, _):
            acnt[pl.ds(kk * _L, _L)] = zero
            return 0
        lax.fori_loop(0, (npt + _L) // _L, init_cnt, 0)
        z16i = jnp.zeros((_L,), jnp.int32)
        for j in range(2 * _BG // _L):
            clist[pl.ds(j * _L, _L)] = z16i
        gdst[pl.ds(_NS * _BG, _L)] = z16i

        def issue_big(start, slot, rsem):
            for j in range(_BG // _L):
                v = clist[pl.ds(start + j * _L, _L)]
                gidx[slot, pl.ds(j * _L, _L)] = v >> 16
                gdst[pl.ds(slot * _BG + j * _L, _L)] = (v & 0xFFFF) - lo
            pltpu.async_copy(y_hbm.at[gidx.at[slot]], ring.at[slot], rsem)

        def drain_big(slot, rsem, count):
            pltpu.make_async_copy(y_hbm.at[gidx.at[slot]],
                                  ring.at[slot], rsem).wait()

            def accum_edge(ei, _):
                d = gdst[pl.ds(slot * _BG + ei, _L)][0]
                for f in range(fb):
                    cs = pl.ds(f * _L, _L)
                    rv = ring[slot, ei, cs]
                    u = accmn[d, cs]
                    mx = jnp.maximum(plsc.bitcast(u & hi_mask, jnp.float32), rv)
                    mn = jnp.minimum(plsc.bitcast(u << 16, jnp.float32), rv)
                    accmn[d, cs] = (
                        ((plsc.bitcast(mx, jnp.int32) + half) & hi_mask)
                        | lax.shift_right_logical(
                            plsc.bitcast(mn, jnp.int32) + half, 16))
                    plsc.addupdate(accs.at[d, cs], rv)
                plsc.addupdate(acnt.at[pl.ds(d, _L)], one0)
                return 0
            lax.fori_loop(0, count, accum_edge, 0)

        def issue_slot(start, k):
            r = k % _NS

            @pl.when(r == 0)
            def _():
                issue_big(start, 0, rsem0)

            @pl.when(r == 1)
            def _():
                issue_big(start, 1, rsem1)

            @pl.when(r == 2)
            def _():
                issue_big(start, 2, rsem2)

        def drain_slot(k, count):
            r = k % _NS

            @pl.when(r == 0)
            def _():
                drain_big(0, rsem0, count)

            @pl.when(r == 1)
            def _():
                drain_big(1, rsem1, count)

            @pl.when(r == 2)
            def _():
                drain_big(2, rsem2, count)

        def pump(ptr, bi):
            nbig = ptr // _BG

            def step(j, b):
                drain_slot(b - 3, jnp.where(b - 3 < _NS, 0, _BG).astype(jnp.int32))
                issue_slot(j * _BG, b)
                return b + 1
            bi = lax.fori_loop(0, nbig, step, bi)
            rem = ptr - nbig * _BG
            for j in range(_BG // _L):
                v = clist[pl.ds(nbig * _BG + j * _L, _L)]
                plsc.store_compressed(clist.at[pl.ds(j * _L, _L)], v,
                                      mask=iota < (rem - j * _L))
            return rem, bi

        def scan_chunk(pev, ptr):
            def scan(v, p):
                w = pev[pl.ds(v * _L, _L)]
                d = w & 0xFFFF
                m = (d >= lo) & (d < hi)
                plsc.store_compressed(clist.at[pl.ds(p, _L)], w, mask=m)
                return p + plsc.all_reduce_population_count(m)[0]
            return lax.fori_loop(0, _CH // _L, scan, ptr)

        def stage(ci, pev):
            pltpu.async_copy(pe_hbm.at[pl.ds(ci * _CH, _CH)], pev, ssem)

        def stage_wait(pev):
            pltpu.make_async_copy(pe_hbm.at[pl.ds(0, _CH)], pev, ssem).wait()

        stage(0, peA)
        issue_big(0, 0, rsem0)
        issue_big(0, 1, rsem1)
        issue_big(0, 2, rsem2)

        def pair(pi, carry):
            ptr, bi = carry
            ci = pi * 2
            stage_wait(peA)
            stage(ci + 1, peB)
            ptr = scan_chunk(peA, ptr)
            ptr, bi = pump(ptr, bi)
            stage_wait(peB)

            @pl.when(ci + 2 < nch)
            def _():
                stage(ci + 2, peA)
            ptr = scan_chunk(peB, ptr)
            ptr, bi = pump(ptr, bi)
            return (ptr, bi)

        ptr, bi = lax.fori_loop(0, nch // 2, pair,
                                (jnp.int32(0), jnp.int32(_NS)))

        drain_slot(bi - 3, jnp.where(bi - 3 < _NS, 0, _BG).astype(jnp.int32))
        drain_slot(bi - 2, jnp.where(bi - 2 < _NS, 0, _BG).astype(jnp.int32))
        drain_slot(bi - 1, jnp.where(bi - 1 < _NS, 0, _BG).astype(jnp.int32))
        issue_slot(0, bi)
        drain_slot(bi, ptr)

        nlast = n - (_NW - 1) * npt

        @pl.when(wid < _NW - 1)
        def _():
            pltpu.sync_copy(accmn, mm_hbm.at[pl.ds(lo, npt)])
            pltpu.sync_copy(accs, sm_hbm.at[pl.ds(lo, npt)])
            pltpu.sync_copy(acnt.at[pl.ds(0, npt)], cnt_hbm.at[pl.ds(lo, npt)])

        @pl.when(wid == _NW - 1)
        def _():
            pltpu.sync_copy(accmn.at[pl.ds(0, nlast)], mm_hbm.at[pl.ds(lo, nlast)])
            pltpu.sync_copy(accs.at[pl.ds(0, nlast)], sm_hbm.at[pl.ds(lo, nlast)])
            pltpu.sync_copy(acnt.at[pl.ds(0, nlast)], cnt_hbm.at[pl.ds(lo, nlast)])

    return k(y, pe)


def _attn_body(sf_ref, mm_ref, sm_ref, cnt_ref, ipw_ref, ipb_ref,
               opw_ref, opb_ref, out_ref):
    c = sf_ref.shape[1]
    sf = sf_ref[...]
    cnt = cnt_ref[...]
    has = cnt > 0.0
    u = mm_ref[...]
    mx = jnp.where(has, lax.bitcast_convert_type(u & (-65536), jnp.float32), 0.0)
    mn = jnp.where(has, lax.bitcast_convert_type(u << 16, jnp.float32), 0.0)
    sm = jnp.where(has, sm_ref[...], 0.0)
    mean = sm / jnp.maximum(cnt, 1.0)
    tokens = [sf, mx, mn, sm, mean]

    ipw = ipw_ref[...]
    ipb = ipb_ref[...]
    dn = (((1,), (1,)), ((), ()))
    big = jnp.concatenate(tokens, axis=0)
    qkv = lax.dot_general(big, ipw, dn, preferred_element_type=jnp.float32) + ipb
    b = sf.shape[0]
    q = [qkv[l * b:(l + 1) * b, 0:c] for l in range(5)]
    k = [qkv[l * b:(l + 1) * b, c:2 * c] for l in range(5)]
    v = [qkv[l * b:(l + 1) * b, 2 * c:3 * c] for l in range(5)]

    scale = 1.0 / jnp.sqrt(jnp.float32(c))
    s = [[jnp.sum(q[l] * k[m], axis=1, keepdims=True) * scale
          for m in range(5)] for l in range(5)]
    w = [jnp.zeros((b, 1), jnp.float32) for _ in range(5)]
    for l in range(5):
        smax = s[l][0]
        for m in range(1, 5):
            smax = jnp.maximum(smax, s[l][m])
        ex = [jnp.exp(s[l][m] - smax) for m in range(5)]
        z = ex[0] + ex[1] + ex[2] + ex[3] + ex[4]
        for m in range(5):
            w[m] = w[m] + ex[m] / z
    ctx = (w[0] * v[0] + w[1] * v[1] + w[2] * v[2] + w[3] * v[3] + w[4] * v[4]) * 0.2
    out = lax.dot_general(ctx, opw_ref[...], dn,
                          preferred_element_type=jnp.float32) + opb_ref[...]
    out_ref[...] = sf + out


def _attention(sf, mm, sm, cnt, ipw, ipb, opw, opb, bn_rows):
    n, c = sf.shape
    grid = n // bn_rows
    full = lambda i: (0, 0)
    blk = pl.BlockSpec((bn_rows, c), lambda i: (i, 0))
    return pl.pallas_call(
        _attn_body,
        grid=(grid,),
        in_specs=[
            blk, blk, blk,
            pl.BlockSpec((bn_rows, 1), lambda i: (i, 0)),
            pl.BlockSpec((3 * c, c), full),
            pl.BlockSpec((1, 3 * c), full),
            pl.BlockSpec((c, c), full),
            pl.BlockSpec((1, c), full),
        ],
        out_specs=blk,
        out_shape=jax.ShapeDtypeStruct((n, c), jnp.float32),
    )(sf, mm, sm, cnt.reshape(n, 1), ipw, ipb.reshape(1, 3 * c),
      opw, opb.reshape(1, c))


def kernel(x, W_neighbor, b_neighbor, W_self, b_self, in_proj_w, in_proj_b,
           out_proj_w, out_proj_b, edge_index):
    n, c = x.shape
    e = edge_index.shape[1]
    pe = (edge_index[0] << 16) | edge_index[1]

    bn_rows = 400 if n % 400 == 0 else n
    y, sf = _proj(x, W_neighbor, b_neighbor, W_self, b_self, bn_rows)
    mm, sm, cnt = _seg_reduce(y, pe, n, e, c)
    return _attention(sf, mm, sm, cnt, in_proj_w, in_proj_b,
                      out_proj_w, out_proj_b, bn_rows)
